# Initial kernel scaffold; baseline (speedup 1.0000x reference)
import jax
import jax.numpy as jnp
from jax.experimental import pallas as pl


def kernel(node_atom, embed_table, W):
    raise NotImplementedError("write your pallas kernel here")


# speedup vs baseline: 1.1133x; 1.1133x over previous
import functools

import jax
import jax.numpy as jnp
from jax import lax
from jax.experimental import pallas as pl
from jax.experimental.pallas import tpu as pltpu
from jax.experimental.pallas import tpu_sc as plsc

NUM_CORES = 2
NUM_SUBCORES = 16
NUM_WORKERS = NUM_CORES * NUM_SUBCORES

EMBED_DIM = 32
IRREPS_DIM = 128
CHUNK = 128


def _fuse_body(tab_ref, w_ref, o_ref):
    o_ref[...] = jnp.dot(
        tab_ref[...], w_ref[...], preferred_element_type=jnp.float32
    ) / jnp.sqrt(jnp.float32(EMBED_DIM))


def _make_sc_gather(n, num_types):
    n_full = n // CHUNK
    tail = n - n_full * CHUNK
    tail_base = n_full * CHUNK
    iters = -(-n_full // NUM_WORKERS)
    tail_worker = NUM_WORKERS - 1

    mesh = plsc.VectorSubcoreMesh(
        core_axis_name="c", subcore_axis_name="s",
        num_cores=NUM_CORES, num_subcores=NUM_SUBCORES,
    )

    scratch = [
        pltpu.VMEM((CHUNK,), jnp.int32),
        pltpu.VMEM((CHUNK, IRREPS_DIM), jnp.float32),
        pltpu.VMEM((CHUNK, EMBED_DIM), jnp.float32),
        pltpu.SemaphoreType.DMA,
        pltpu.SemaphoreType.DMA,
    ]
    if tail:
        scratch += [
            pltpu.VMEM((tail,), jnp.int32),
            pltpu.VMEM((tail, IRREPS_DIM), jnp.float32),
            pltpu.VMEM((tail, EMBED_DIM), jnp.float32),
        ]

    @functools.partial(
        pl.kernel,
        out_type=(
            jax.ShapeDtypeStruct((n, IRREPS_DIM), jnp.float32),
            jax.ShapeDtypeStruct((n, EMBED_DIM), jnp.float32),
            jax.ShapeDtypeStruct((n, EMBED_DIM), jnp.float32),
        ),
        mesh=mesh,
        scratch_types=scratch,
        compiler_params=pltpu.CompilerParams(use_tc_tiling_on_sc=False),
    )
    def sc_gather(idx_hbm, tab_hbm, fused_hbm, ne_hbm, d1_hbm, d2_hbm,
                  idx_v, ne_buf, d_buf, sem_a, sem_b, *tail_scratch):
        w = lax.axis_index("s") * NUM_CORES + lax.axis_index("c")

        def do_chunk(base, idx_ref, ne_ref, d_ref, sz):
            pltpu.sync_copy(idx_hbm.at[pl.ds(base, sz)], idx_ref)
            cp_a = pltpu.async_copy(fused_hbm.at[idx_ref], ne_ref, sem_a)
            cp_b = pltpu.async_copy(tab_hbm.at[idx_ref], d_ref, sem_b)
            cp_a.wait()
            cp_b.wait()
            pltpu.sync_copy(ne_ref, ne_hbm.at[pl.ds(base, sz)])
            pltpu.sync_copy(d_ref, d1_hbm.at[pl.ds(base, sz)])
            pltpu.sync_copy(d_ref, d2_hbm.at[pl.ds(base, sz)])

        def loop_body(i, carry):
            cid = w + NUM_WORKERS * i

            @pl.when(cid < n_full)
            def _():
                do_chunk(cid * CHUNK, idx_v, ne_buf, d_buf, CHUNK)

            return carry

        lax.fori_loop(0, iters, loop_body, 0)

        if tail:
            idx_t, ne_t, d_t = tail_scratch

            @pl.when(w == tail_worker)
            def _():
                do_chunk(tail_base, idx_t, ne_t, d_t, tail)

    return sc_gather


def kernel(node_atom, embed_table, W):
    node_atom = node_atom.astype(jnp.int32)
    n = node_atom.shape[0]
    num_types = embed_table.shape[0]

    fused = pl.pallas_call(
        _fuse_body,
        out_shape=jax.ShapeDtypeStruct((num_types, IRREPS_DIM), jnp.float32),
    )(embed_table, W)

    sc_gather = _make_sc_gather(n, num_types)
    node_embedding, atom_attr, atom_dense = sc_gather(node_atom, embed_table, fused)
    return (node_embedding, atom_attr, atom_dense)


# speedup vs baseline: 1.2384x; 1.1124x over previous
import functools

import jax
import jax.numpy as jnp
from jax import lax
from jax.experimental import pallas as pl
from jax.experimental.pallas import tpu as pltpu
from jax.experimental.pallas import tpu_sc as plsc

NUM_CORES = 2
NUM_SUBCORES = 16
NUM_WORKERS = NUM_CORES * NUM_SUBCORES

EMBED_DIM = 32
IRREPS_DIM = 128
CHUNK = 128
DENSE_BLK = 2000


def _fuse_body(tab_ref, w_ref, o_ref):
    o_ref[...] = jnp.dot(
        tab_ref[...], w_ref[...], preferred_element_type=jnp.float32
    ) / jnp.sqrt(jnp.float32(EMBED_DIM))


def _dense_body(idx_ref, tab_ref, o1_ref, o2_ref):
    idx = idx_ref[0]
    num_types = tab_ref.shape[0]
    onehot = (idx == lax.broadcasted_iota(
        jnp.int32, (num_types, 1), 0)).astype(jnp.float32)
    d = lax.dot_general(
        onehot, tab_ref[...], (((0,), (0,)), ((), ())),
        preferred_element_type=jnp.float32)
    o1_ref[...] = d
    o2_ref[...] = d


def _make_sc_gather(n):
    n_full = n // CHUNK
    tail = n - n_full * CHUNK
    tail_base = n_full * CHUNK
    iters = -(-n_full // NUM_WORKERS)
    tail_worker = NUM_WORKERS - 1

    mesh = plsc.VectorSubcoreMesh(
        core_axis_name="c", subcore_axis_name="s",
        num_cores=NUM_CORES, num_subcores=NUM_SUBCORES,
    )

    scratch = [
        pltpu.VMEM((CHUNK,), jnp.int32),
        pltpu.VMEM((CHUNK, IRREPS_DIM), jnp.float32),
        pltpu.SemaphoreType.DMA,
    ]
    if tail:
        scratch += [
            pltpu.VMEM((tail,), jnp.int32),
            pltpu.VMEM((tail, IRREPS_DIM), jnp.float32),
        ]

    @functools.partial(
        pl.kernel,
        out_type=jax.ShapeDtypeStruct((n, IRREPS_DIM), jnp.float32),
        mesh=mesh,
        scratch_types=scratch,
    )
    def sc_gather(idx_hbm, fused_hbm, ne_hbm, idx_v, ne_buf, sem, *tail_scratch):
        w = lax.axis_index("s") * NUM_CORES + lax.axis_index("c")

        def do_chunk(base, idx_ref, ne_ref, sz):
            pltpu.sync_copy(idx_hbm.at[pl.ds(base, sz)], idx_ref)
            pltpu.async_copy(fused_hbm.at[idx_ref], ne_ref, sem).wait()
            pltpu.sync_copy(ne_ref, ne_hbm.at[pl.ds(base, sz)])

        def loop_body(i, carry):
            cid = w + NUM_WORKERS * i

            @pl.when(cid < n_full)
            def _():
                do_chunk(cid * CHUNK, idx_v, ne_buf, CHUNK)

            return carry

        lax.fori_loop(0, iters, loop_body, 0)

        if tail:
            idx_t, ne_t = tail_scratch

            @pl.when(w == tail_worker)
            def _():
                do_chunk(tail_base, idx_t, ne_t, tail)

    return sc_gather


def kernel(node_atom, embed_table, W):
    node_atom = node_atom.astype(jnp.int32)
    n = node_atom.shape[0]
    num_types = embed_table.shape[0]

    fused = pl.pallas_call(
        _fuse_body,
        out_shape=jax.ShapeDtypeStruct((num_types, IRREPS_DIM), jnp.float32),
    )(embed_table, W)

    node_embedding = _make_sc_gather(n)(node_atom, fused)

    blk = DENSE_BLK if n % DENSE_BLK == 0 else n
    grid = n // blk
    idx3d = node_atom.reshape(grid, 1, blk)
    atom_attr, atom_dense = pl.pallas_call(
        _dense_body,
        grid=(grid,),
        in_specs=[
            pl.BlockSpec((1, 1, blk), lambda i: (i, 0, 0)),
            pl.BlockSpec((num_types, EMBED_DIM), lambda i: (0, 0)),
        ],
        out_specs=[
            pl.BlockSpec((blk, EMBED_DIM), lambda i: (i, 0)),
            pl.BlockSpec((blk, EMBED_DIM), lambda i: (i, 0)),
        ],
        out_shape=[
            jax.ShapeDtypeStruct((n, EMBED_DIM), jnp.float32),
            jax.ShapeDtypeStruct((n, EMBED_DIM), jnp.float32),
        ],
    )(idx3d, embed_table)

    return (node_embedding, atom_attr, atom_dense)


# speedup vs baseline: 1.2406x; 1.0018x over previous
import functools

import jax
import jax.numpy as jnp
from jax import lax
from jax.experimental import pallas as pl
from jax.experimental.pallas import tpu as pltpu
from jax.experimental.pallas import tpu_sc as plsc

NUM_CORES = 2
NUM_SUBCORES = 16
NUM_WORKERS = NUM_CORES * NUM_SUBCORES

EMBED_DIM = 32
IRREPS_DIM = 128
CHUNK = 128
DENSE_BLK = 2000


def _fuse_body(tab_ref, w_ref, o_ref):
    o_ref[...] = jnp.dot(
        tab_ref[...], w_ref[...], preferred_element_type=jnp.float32
    ) / jnp.sqrt(jnp.float32(EMBED_DIM))


def _dense_body(idx_ref, tab_ref, o1_ref, o2_ref):
    idx = idx_ref[0]
    num_types = tab_ref.shape[0]
    onehot = (idx == lax.broadcasted_iota(
        jnp.int32, (num_types, 1), 0)).astype(jnp.float32)
    d = lax.dot_general(
        onehot, tab_ref[...], (((0,), (0,)), ((), ())),
        preferred_element_type=jnp.float32)
    o1_ref[...] = d
    o2_ref[...] = d


def _make_sc_gather(n):
    n_full = n // CHUNK
    tail = n - n_full * CHUNK
    tail_base = n_full * CHUNK
    iters = -(-n_full // NUM_WORKERS)
    tail_worker = NUM_WORKERS - 1

    mesh = plsc.VectorSubcoreMesh(
        core_axis_name="c", subcore_axis_name="s",
        num_cores=NUM_CORES, num_subcores=NUM_SUBCORES,
    )

    scratch = [
        pltpu.VMEM((CHUNK,), jnp.int32),
        pltpu.VMEM((CHUNK, IRREPS_DIM), jnp.float32),
        pltpu.SemaphoreType.DMA,
    ]
    if tail:
        scratch += [
            pltpu.VMEM((tail,), jnp.int32),
            pltpu.VMEM((tail, IRREPS_DIM), jnp.float32),
        ]

    @functools.partial(
        pl.kernel,
        out_type=jax.ShapeDtypeStruct((n, IRREPS_DIM), jnp.float32),
        mesh=mesh,
        scratch_types=scratch,
        compiler_params=pltpu.CompilerParams(use_tc_tiling_on_sc=False),
    )
    def sc_gather(idx_hbm, fused_hbm, ne_hbm, idx_v, ne_buf, sem, *tail_scratch):
        w = lax.axis_index("s") * NUM_CORES + lax.axis_index("c")

        def do_chunk(base, idx_ref, ne_ref, sz):
            pltpu.sync_copy(idx_hbm.at[pl.ds(base, sz)], idx_ref)
            pltpu.async_copy(fused_hbm.at[idx_ref], ne_ref, sem).wait()
            pltpu.sync_copy(ne_ref, ne_hbm.at[pl.ds(base, sz)])

        def loop_body(i, carry):
            cid = w + NUM_WORKERS * i

            @pl.when(cid < n_full)
            def _():
                do_chunk(cid * CHUNK, idx_v, ne_buf, CHUNK)

            return carry

        lax.fori_loop(0, iters, loop_body, 0)

        if tail:
            idx_t, ne_t = tail_scratch

            @pl.when(w == tail_worker)
            def _():
                do_chunk(tail_base, idx_t, ne_t, tail)

    return sc_gather


def kernel(node_atom, embed_table, W):
    node_atom = node_atom.astype(jnp.int32)
    n = node_atom.shape[0]
    num_types = embed_table.shape[0]

    fused = pl.pallas_call(
        _fuse_body,
        out_shape=jax.ShapeDtypeStruct((num_types, IRREPS_DIM), jnp.float32),
    )(embed_table, W)

    node_embedding = _make_sc_gather(n)(node_atom, fused)

    blk = DENSE_BLK if n % DENSE_BLK == 0 else n
    grid = n // blk
    idx3d = node_atom.reshape(grid, 1, blk)
    atom_attr, atom_dense = pl.pallas_call(
        _dense_body,
        grid=(grid,),
        in_specs=[
            pl.BlockSpec((1, 1, blk), lambda i: (i, 0, 0)),
            pl.BlockSpec((num_types, EMBED_DIM), lambda i: (0, 0)),
        ],
        out_specs=[
            pl.BlockSpec((blk, EMBED_DIM), lambda i: (i, 0)),
            pl.BlockSpec((blk, EMBED_DIM), lambda i: (i, 0)),
        ],
        out_shape=[
            jax.ShapeDtypeStruct((n, EMBED_DIM), jnp.float32),
            jax.ShapeDtypeStruct((n, EMBED_DIM), jnp.float32),
        ],
    )(idx3d, embed_table)

    return (node_embedding, atom_attr, atom_dense)
